# baseline (device time: 405980 ns/iter reference)
import jax
import jax.numpy as jnp
from jax import lax
from jax.experimental import pallas as pl
from jax.experimental.pallas import tpu as pltpu

N_DEV = 32
B = 2
SQ = 128
SKV = 128
H_LOC = 4
DH = 64
D_MODEL = 512
D_LOC = H_LOC * DH


def kernel(x, Wq, K_ext, V_ext, Wo):
    my = lax.axis_index("i")
    wq_loc = lax.dynamic_slice_in_dim(Wq, my * D_LOC, D_LOC, axis=1)
    wo_loc = lax.dynamic_slice_in_dim(Wo, my * D_LOC, D_LOC, axis=0)

    def body(x_ref, wq_ref, k_ref, v_ref, wo_ref, out_ref,
             comm_ref, send_sems, recv_sems, ack_sem):
        my_pos = lax.axis_index("i")
        left = lax.rem(my_pos - 1 + N_DEV, N_DEV)
        right = lax.rem(my_pos + 1, N_DEV)

        barrier_sem = pltpu.get_barrier_semaphore()
        for nbr in (left, right):
            pl.semaphore_signal(barrier_sem, inc=1, device_id=(nbr,),
                                device_id_type=pl.DeviceIdType.MESH)
        pl.semaphore_wait(barrier_sem, 2)

        rowblk = lax.broadcasted_iota(jnp.int32, (SQ, SKV), 0) // 64
        colblk = lax.broadcasted_iota(jnp.int32, (SQ, SKV), 1) // 64
        mask = colblk <= rowblk

        for b in range(B):
            xb = x_ref[b]
            qb = jnp.dot(xb, wq_ref[...],
                         preferred_element_type=jnp.float32)
            ctxs = []
            for h in range(H_LOC):
                qh = qb[:, h * DH:(h + 1) * DH]
                kh = k_ref[b, :, h, :]
                vh = v_ref[b, :, h, :]
                s = lax.dot_general(
                    qh, kh, (((1,), (1,)), ((), ())),
                    preferred_element_type=jnp.float32) * 0.125
                s = jnp.where(mask, s, -1e9)
                m = jnp.max(s, axis=1, keepdims=True)
                w = jnp.exp(s - m)
                w = w / jnp.sum(w, axis=1, keepdims=True)
                ctxs.append(jnp.dot(w, vh,
                                    preferred_element_type=jnp.float32))
            ctx = jnp.concatenate(ctxs, axis=1)
            part = jnp.dot(ctx, wo_ref[...],
                           preferred_element_type=jnp.float32)
            out_ref[b] = part
            comm_ref[0, b] = part

        for h in range(N_DEV - 1):
            send_slot = h % 2
            recv_slot = (h + 1) % 2
            if h >= 1:
                pl.semaphore_wait(ack_sem, 1)
            rdma = pltpu.make_async_remote_copy(
                src_ref=comm_ref.at[send_slot],
                dst_ref=comm_ref.at[recv_slot],
                send_sem=send_sems.at[send_slot],
                recv_sem=recv_sems.at[recv_slot],
                device_id=(right,),
                device_id_type=pl.DeviceIdType.MESH,
            )
            rdma.start()
            rdma.wait()
            out_ref[...] += comm_ref[recv_slot]
            if h < N_DEV - 2:
                pl.semaphore_signal(ack_sem, inc=1, device_id=(left,),
                                    device_id_type=pl.DeviceIdType.MESH)

    return pl.pallas_call(
        body,
        out_shape=jax.ShapeDtypeStruct((B, SQ, D_MODEL), jnp.float32),
        in_specs=[pl.BlockSpec(memory_space=pltpu.VMEM)] * 5,
        out_specs=pl.BlockSpec(memory_space=pltpu.VMEM),
        scratch_shapes=[
            pltpu.VMEM((2, B, SQ, D_MODEL), jnp.float32),
            pltpu.SemaphoreType.DMA((2,)),
            pltpu.SemaphoreType.DMA((2,)),
            pltpu.SemaphoreType.REGULAR,
        ],
        compiler_params=pltpu.CompilerParams(collective_id=0),
    )(x, wq_loc, K_ext, V_ext, wo_loc)


# device time: 45525 ns/iter; 8.9177x vs baseline; 8.9177x over previous
import jax
import jax.numpy as jnp
from jax import lax
from jax.experimental import pallas as pl
from jax.experimental.pallas import tpu as pltpu

N_DEV = 32
LOG2_N = 5
B = 2
SQ = 128
SKV = 128
H_LOC = 4
DH = 64
D_MODEL = 512
D_LOC = H_LOC * DH
ROWS = B * SQ
HALVES = [ROWS // (1 << (r + 1)) for r in range(LOG2_N)]


def kernel(x, Wq, K_ext, V_ext, Wo):
    my = lax.axis_index("i")
    wq_loc = lax.dynamic_slice_in_dim(Wq, my * D_LOC, D_LOC, axis=1)
    wo_loc = lax.dynamic_slice_in_dim(Wo, my * D_LOC, D_LOC, axis=0)

    def body(x_ref, wq_ref, k_ref, v_ref, wo_ref, out_ref,
             acc_ref, rs_ref, send_sems, recv_sems):
        my_pos = lax.axis_index("i")

        barrier_sem = pltpu.get_barrier_semaphore()
        for d in range(N_DEV - 1):
            tgt = lax.rem(my_pos + 1 + d, N_DEV)
            pl.semaphore_signal(barrier_sem, inc=1, device_id=(tgt,),
                                device_id_type=pl.DeviceIdType.MESH)

        rowblk = lax.broadcasted_iota(jnp.int32, (SQ, SKV), 0) // 64
        colblk = lax.broadcasted_iota(jnp.int32, (SQ, SKV), 1) // 64
        mask = colblk <= rowblk

        for b in range(B):
            xb = x_ref[b]
            qb = jnp.dot(xb, wq_ref[...],
                         preferred_element_type=jnp.float32)
            ctxs = []
            for h in range(H_LOC):
                qh = qb[:, h * DH:(h + 1) * DH]
                kh = k_ref[b, :, h, :]
                vh = v_ref[b, :, h, :]
                s = lax.dot_general(
                    qh, kh, (((1,), (1,)), ((), ())),
                    preferred_element_type=jnp.float32) * 0.125
                s = jnp.where(mask, s, -1e9)
                m = jnp.max(s, axis=1, keepdims=True)
                w = jnp.exp(s - m)
                w = w / jnp.sum(w, axis=1, keepdims=True)
                ctxs.append(jnp.dot(w, vh,
                                    preferred_element_type=jnp.float32))
            ctx = jnp.concatenate(ctxs, axis=1)
            part = jnp.dot(ctx, wo_ref[...],
                           preferred_element_type=jnp.float32)
            acc_ref[pl.ds(b * SQ, SQ)] = part

        pl.semaphore_wait(barrier_sem, N_DEV - 1)

        off = jnp.int32(0)
        for r in range(LOG2_N):
            half = HALVES[r]
            bit = lax.bitwise_and(lax.shift_right_logical(my_pos, r), 1)
            partner = lax.bitwise_xor(my_pos, 1 << r)
            send_off = off + (1 - bit) * half
            keep_off = off + bit * half
            rdma = pltpu.make_async_remote_copy(
                src_ref=acc_ref.at[pl.ds(send_off, half)],
                dst_ref=rs_ref.at[r, pl.ds(0, half)],
                send_sem=send_sems.at[r],
                recv_sem=recv_sems.at[r],
                device_id=(partner,),
                device_id_type=pl.DeviceIdType.MESH,
            )
            rdma.start()
            rdma.wait()
            acc_ref[pl.ds(keep_off, half)] = (
                acc_ref[pl.ds(keep_off, half)] + rs_ref[r, pl.ds(0, half)]
            )
            off = keep_off

        for r in range(LOG2_N - 1, -1, -1):
            seg = HALVES[r]
            bit = lax.bitwise_and(lax.shift_right_logical(my_pos, r), 1)
            partner = lax.bitwise_xor(my_pos, 1 << r)
            rdma = pltpu.make_async_remote_copy(
                src_ref=acc_ref.at[pl.ds(off, seg)],
                dst_ref=acc_ref.at[pl.ds(off, seg)],
                send_sem=send_sems.at[LOG2_N + r],
                recv_sem=recv_sems.at[LOG2_N + r],
                device_id=(partner,),
                device_id_type=pl.DeviceIdType.MESH,
            )
            rdma.start()
            rdma.wait()
            off = off - bit * seg

        for b in range(B):
            out_ref[b] = acc_ref[pl.ds(b * SQ, SQ)]

    return pl.pallas_call(
        body,
        out_shape=jax.ShapeDtypeStruct((B, SQ, D_MODEL), jnp.float32),
        in_specs=[pl.BlockSpec(memory_space=pltpu.VMEM)] * 5,
        out_specs=pl.BlockSpec(memory_space=pltpu.VMEM),
        scratch_shapes=[
            pltpu.VMEM((ROWS, D_MODEL), jnp.float32),
            pltpu.VMEM((LOG2_N, ROWS // 2, D_MODEL), jnp.float32),
            pltpu.SemaphoreType.DMA((2 * LOG2_N,)),
            pltpu.SemaphoreType.DMA((2 * LOG2_N,)),
        ],
        compiler_params=pltpu.CompilerParams(collective_id=0),
    )(x, wq_loc, K_ext, V_ext, wo_loc)


# device time: 29143 ns/iter; 13.9306x vs baseline; 1.5621x over previous
import jax
import jax.numpy as jnp
from jax import lax
from jax.experimental import pallas as pl
from jax.experimental.pallas import tpu as pltpu

N_DEV = 32
B = 2
SQ = 128
SKV = 128
H_LOC = 4
DH = 64
D_MODEL = 512
D_LOC = H_LOC * DH
ROWS = B * SQ
SEG = ROWS // N_DEV
SEG_PER_B = SQ // SEG


def kernel(x, Wq, K_ext, V_ext, Wo):
    my = lax.axis_index("i")
    wq_loc = lax.dynamic_slice_in_dim(Wq, my * D_LOC, D_LOC, axis=1)
    wo_loc = lax.dynamic_slice_in_dim(Wo, my * D_LOC, D_LOC, axis=0)
    k2 = K_ext.reshape(B, SKV, D_LOC)
    v2 = V_ext.reshape(B, SKV, D_LOC)

    def body(x_ref, wq_ref, k_ref, v_ref, wo_ref, out_ref,
             acc_ref, rs_ref, ph1_send, ph1_recv, ph2_send, ph2_recv):
        my_pos = lax.axis_index("i")

        barrier_sem = pltpu.get_barrier_semaphore()
        for d in range(N_DEV - 1):
            tgt = lax.rem(my_pos + 1 + d, N_DEV)
            pl.semaphore_signal(barrier_sem, inc=1, device_id=(tgt,),
                                device_id_type=pl.DeviceIdType.MESH)

        rowblk = lax.broadcasted_iota(jnp.int32, (SQ, SKV), 0) // 64
        colblk = lax.broadcasted_iota(jnp.int32, (SQ, SKV), 1) // 64
        mask = colblk <= rowblk

        def compute_b(b):
            xb = x_ref[b]
            qb = jnp.dot(xb, wq_ref[...],
                         preferred_element_type=jnp.float32)
            kb = k_ref[b]
            vb = v_ref[b]
            ctxs = []
            for h in range(H_LOC):
                qh = qb[:, h * DH:(h + 1) * DH]
                kh = kb[:, h * DH:(h + 1) * DH]
                vh = vb[:, h * DH:(h + 1) * DH]
                s = lax.dot_general(
                    qh, kh, (((1,), (1,)), ((), ())),
                    preferred_element_type=jnp.float32) * 0.125
                s = jnp.where(mask, s, -1e9)
                m = jnp.max(s, axis=1, keepdims=True)
                w = jnp.exp(s - m)
                w = w / jnp.sum(w, axis=1, keepdims=True)
                ctxs.append(jnp.dot(w, vh,
                                    preferred_element_type=jnp.float32))
            ctx = jnp.concatenate(ctxs, axis=1)
            acc_ref[pl.ds(b * SQ, SQ)] = jnp.dot(
                ctx, wo_ref[...], preferred_element_type=jnp.float32)

        def send_group(seg_lo, seg_hi):
            for s in range(seg_lo, seg_hi):
                @pl.when(my_pos != s)
                def _():
                    rdma = pltpu.make_async_remote_copy(
                        src_ref=acc_ref.at[pl.ds(s * SEG, SEG)],
                        dst_ref=rs_ref.at[pl.ds(my_pos * SEG, SEG)],
                        send_sem=ph1_send,
                        recv_sem=ph1_recv,
                        device_id=(s,),
                        device_id_type=pl.DeviceIdType.MESH,
                    )
                    rdma.start()

        compute_b(0)
        compute_b(1)
        pl.semaphore_wait(barrier_sem, N_DEV - 1)
        send_group(0, N_DEV)

        rs_ref[pl.ds(my_pos * SEG, SEG)] = acc_ref[pl.ds(my_pos * SEG, SEG)]

        wait_ph1 = pltpu.make_async_remote_copy(
            src_ref=rs_ref.at[pl.ds(0, (N_DEV - 1) * SEG)],
            dst_ref=rs_ref.at[pl.ds(0, (N_DEV - 1) * SEG)],
            send_sem=ph1_send, recv_sem=ph1_recv,
            device_id=(my_pos,), device_id_type=pl.DeviceIdType.MESH,
        )
        wait_ph1.wait_recv()

        red = jnp.sum(
            rs_ref[...].reshape(N_DEV, SEG, D_MODEL), axis=0)
        acc_ref[pl.ds(my_pos * SEG, SEG)] = red

        wait_ph1.wait_send()

        for s in range(N_DEV):
            @pl.when(my_pos != s)
            def _():
                rdma = pltpu.make_async_remote_copy(
                    src_ref=acc_ref.at[pl.ds(my_pos * SEG, SEG)],
                    dst_ref=acc_ref.at[pl.ds(my_pos * SEG, SEG)],
                    send_sem=ph2_send,
                    recv_sem=ph2_recv,
                    device_id=(s,),
                    device_id_type=pl.DeviceIdType.MESH,
                )
                rdma.start()

        wait_ph2 = pltpu.make_async_remote_copy(
            src_ref=rs_ref.at[pl.ds(0, (N_DEV - 1) * SEG)],
            dst_ref=rs_ref.at[pl.ds(0, (N_DEV - 1) * SEG)],
            send_sem=ph2_send, recv_sem=ph2_recv,
            device_id=(my_pos,), device_id_type=pl.DeviceIdType.MESH,
        )
        wait_ph2.wait_recv()

        for b in range(B):
            out_ref[b] = acc_ref[pl.ds(b * SQ, SQ)]

        wait_ph2.wait_send()

    return pl.pallas_call(
        body,
        out_shape=jax.ShapeDtypeStruct((B, SQ, D_MODEL), jnp.float32),
        in_specs=[pl.BlockSpec(memory_space=pltpu.VMEM)] * 5,
        out_specs=pl.BlockSpec(memory_space=pltpu.VMEM),
        scratch_shapes=[
            pltpu.VMEM((ROWS, D_MODEL), jnp.float32),
            pltpu.VMEM((ROWS, D_MODEL), jnp.float32),
            pltpu.SemaphoreType.DMA,
            pltpu.SemaphoreType.DMA,
            pltpu.SemaphoreType.DMA,
            pltpu.SemaphoreType.DMA,
        ],
        compiler_params=pltpu.CompilerParams(collective_id=0),
    )(x, wq_loc, k2, v2, wo_loc)
